# baseline (speedup 1.0000x reference)
import functools
import math

import jax
import jax.numpy as jnp
import numpy as np
from jax import lax
from jax.experimental import pallas as pl
from jax.experimental.pallas import tpu as pltpu
from jax.experimental.pallas import tpu_sc as plsc

_VOCAB = 100000
_MASK = 99999
_GAMMA = 1.0
_P_UNIFORM = max(math.exp(-20.0), 0.0)
_LOG_B = max(
    _GAMMA * math.log(2.0) + math.log(_P_UNIFORM) - math.log(1.0 - _P_UNIFORM),
    -20.0,
)
_B_CONST = math.exp(_LOG_B)

_KEY_HI = np.uint32(0)
_KEY_LO = np.uint32(42)
_KS2 = np.uint32(int(_KEY_HI) ^ int(_KEY_LO) ^ 0x1BD11BDA)
_ROT_A = (13, 15, 26, 6)
_ROT_B = (17, 29, 16, 24)
_TINY = np.float32(np.finfo(np.float32).tiny)
_SPAN = np.float32(np.float32(1.0) - _TINY)

_NC, _NS, _L = 2, 16, 16


def _rotl(x, r):
    r = np.uint32(r)
    return (x << r) | (x >> np.uint32(32 - r))


def _threefry2x32(ctr_lo):
    ks = (_KEY_HI, _KEY_LO, _KS2)
    x0 = jnp.full_like(ctr_lo, ks[0])
    x1 = ctr_lo + ks[1]

    def rounds(x0, x1, rots):
        for r in rots:
            x0 = x0 + x1
            x1 = _rotl(x1, r)
            x1 = x0 ^ x1
        return x0, x1

    x0, x1 = rounds(x0, x1, _ROT_A)
    x0 = x0 + ks[1]
    x1 = x1 + ks[2] + np.uint32(1)
    x0, x1 = rounds(x0, x1, _ROT_B)
    x0 = x0 + ks[2]
    x1 = x1 + ks[0] + np.uint32(2)
    x0, x1 = rounds(x0, x1, _ROT_A)
    x0 = x0 + ks[0]
    x1 = x1 + ks[1] + np.uint32(3)
    x0, x1 = rounds(x0, x1, _ROT_B)
    x0 = x0 + ks[1]
    x1 = x1 + ks[2] + np.uint32(4)
    x0, x1 = rounds(x0, x1, _ROT_A)
    x0 = x0 + ks[2]
    x1 = x1 + ks[0] + np.uint32(5)
    return x0 ^ x1


@functools.partial(
    pl.kernel,
    out_type=jax.ShapeDtypeStruct((_NC * _NS * _L,), jnp.uint32),
    mesh=plsc.VectorSubcoreMesh(core_axis_name="c", subcore_axis_name="s"),
    scratch_types=[
        pltpu.VMEM((_L,), jnp.uint32),
        pltpu.VMEM((_L,), jnp.uint32),
    ],
)
def _sc_bits(ctr_hbm, out_hbm, ctr_v, bits_v):
    wid = lax.axis_index("s") * _NC + lax.axis_index("c")
    base = wid * _L
    pltpu.sync_copy(ctr_hbm.at[pl.ds(base, _L)], ctr_v)
    bits_v[...] = _threefry2x32(ctr_v[...])
    pltpu.sync_copy(bits_v, out_hbm.at[pl.ds(base, _L)])


def _gumbel_from_bits(bits):
    float_bits = (bits >> np.uint32(9)) | np.uint32(0x3F800000)
    floats = jax.lax.bitcast_convert_type(float_bits, jnp.float32) - jnp.float32(1.0)
    u = jnp.maximum(_TINY, floats * _SPAN + _TINY)
    return -jnp.log(-jnp.log(u))


def _decide_body(bits_ref, ids_ref, t_ref, z_ref):
    bits = bits_ref[...]
    ids = ids_ref[...]
    t = t_ref[...]

    g_tok = _gumbel_from_bits(bits[0:2, :])
    g_msk = _gumbel_from_bits(bits[2:4, :])

    one = jnp.float32(1.0)
    zero = jnp.float32(0.0)
    c_t = jnp.power(t, jnp.float32(0.5)) * jnp.power(one - t, jnp.float32(0.5)) * jnp.float32(_B_CONST)
    big_c = jnp.maximum(one + c_t, jnp.float32(1e-4))
    alpha = (one - t) / big_c
    unif = jnp.float32(np.float32(1.0) / np.float32(_VOCAB - 1))
    beta_tok = (t * zero + c_t * unif) / big_c
    beta_msk = (t * one + c_t * zero) / big_c

    is_mask_tok = ids == _MASK
    p_tok = jnp.where(is_mask_tok, one * alpha + beta_msk, one * alpha + beta_tok)
    p_msk = jnp.where(is_mask_tok, p_tok, zero * alpha + beta_msk)
    eps = jnp.float32(1e-12)
    s_tok = jnp.log(p_tok + eps) + g_tok
    s_msk = jnp.log(p_msk + eps) + g_msk

    z = jnp.where(is_mask_tok, _MASK, jnp.where(s_tok >= s_msk, ids, _MASK))
    z_ref[...] = z.astype(jnp.int32)


@functools.partial(jax.jit, static_argnames=("interpret",))
def kernel(input_ids, t, *, interpret=False):
    b, s = input_ids.shape
    n = b * s
    ids_flat = input_ids.astype(jnp.int32).reshape(n)
    rows = jnp.arange(n, dtype=jnp.int32)
    ctr = jnp.concatenate([rows * _VOCAB + ids_flat, rows * _VOCAB + _MASK])
    bits = _sc_bits(ctr.astype(jnp.uint32))

    ids2 = ids_flat.reshape(2, n // 2)
    t2 = jnp.broadcast_to(t.astype(jnp.float32)[:, None], (b, s)).reshape(2, n // 2)
    z = pl.pallas_call(
        _decide_body,
        out_shape=jax.ShapeDtypeStruct((2, n // 2), jnp.int32),
        interpret=interpret,
    )(bits.reshape(4, n // 2), ids2, t2)
    return z.reshape(b, s)


# speedup vs baseline: 4.6762x; 4.6762x over previous
import functools
import math

import jax
import jax.numpy as jnp
import numpy as np
from jax.experimental import pallas as pl

_VOCAB = 100000
_MASK = 99999
_GAMMA = 1.0
_P_UNIFORM = max(math.exp(-20.0), 0.0)
_LOG_B = max(
    _GAMMA * math.log(2.0) + math.log(_P_UNIFORM) - math.log(1.0 - _P_UNIFORM),
    -20.0,
)
_B_CONST = math.exp(_LOG_B)

_KEY_HI = np.uint32(0)
_KEY_LO = np.uint32(42)
_KS2 = np.uint32(int(_KEY_HI) ^ int(_KEY_LO) ^ 0x1BD11BDA)
_ROT_A = (13, 15, 26, 6)
_ROT_B = (17, 29, 16, 24)
_TINY = np.float32(np.finfo(np.float32).tiny)
_SPAN = np.float32(np.float32(1.0) - _TINY)


def _rotl(x, r):
    r = np.uint32(r)
    return (x << r) | (x >> np.uint32(32 - r))


def _threefry2x32(ctr_lo):
    ks = (_KEY_HI, _KEY_LO, _KS2)
    x0 = jnp.full_like(ctr_lo, ks[0])
    x1 = ctr_lo + ks[1]

    def rounds(x0, x1, rots):
        for r in rots:
            x0 = x0 + x1
            x1 = _rotl(x1, r)
            x1 = x0 ^ x1
        return x0, x1

    x0, x1 = rounds(x0, x1, _ROT_A)
    x0 = x0 + ks[1]
    x1 = x1 + ks[2] + np.uint32(1)
    x0, x1 = rounds(x0, x1, _ROT_B)
    x0 = x0 + ks[2]
    x1 = x1 + ks[0] + np.uint32(2)
    x0, x1 = rounds(x0, x1, _ROT_A)
    x0 = x0 + ks[0]
    x1 = x1 + ks[1] + np.uint32(3)
    x0, x1 = rounds(x0, x1, _ROT_B)
    x0 = x0 + ks[1]
    x1 = x1 + ks[2] + np.uint32(4)
    x0, x1 = rounds(x0, x1, _ROT_A)
    x0 = x0 + ks[2]
    x1 = x1 + ks[0] + np.uint32(5)
    return x0 ^ x1


def _gumbel_from_bits(bits):
    float_bits = (bits >> np.uint32(9)) | np.uint32(0x3F800000)
    floats = jax.lax.bitcast_convert_type(float_bits, jnp.float32) - jnp.float32(1.0)
    u = jnp.maximum(_TINY, floats * _SPAN + _TINY)
    return -jnp.log(-jnp.log(u))


def _sample_body(ids_ref, t_ref, z_ref):
    ids = ids_ref[...]
    t = t_ref[...]

    row = (
        jax.lax.broadcasted_iota(jnp.int32, ids.shape, 0) * ids.shape[1]
        + jax.lax.broadcasted_iota(jnp.int32, ids.shape, 1)
    )
    base = row * _VOCAB
    bits_tok = _threefry2x32((base + ids).astype(jnp.uint32))
    bits_msk = _threefry2x32((base + _MASK).astype(jnp.uint32))
    g_tok = _gumbel_from_bits(bits_tok)
    g_msk = _gumbel_from_bits(bits_msk)

    one = jnp.float32(1.0)
    zero = jnp.float32(0.0)
    c_t = jnp.power(t, jnp.float32(0.5)) * jnp.power(one - t, jnp.float32(0.5)) * jnp.float32(_B_CONST)
    big_c = jnp.maximum(one + c_t, jnp.float32(1e-4))
    alpha = (one - t) / big_c
    unif = jnp.float32(np.float32(1.0) / np.float32(_VOCAB - 1))
    beta_tok = (t * zero + c_t * unif) / big_c
    beta_msk = (t * one + c_t * zero) / big_c

    is_mask_tok = ids == _MASK
    p_tok = jnp.where(is_mask_tok, one * alpha + beta_msk, one * alpha + beta_tok)
    p_msk = jnp.where(is_mask_tok, p_tok, zero * alpha + beta_msk)
    eps = jnp.float32(1e-12)
    s_tok = jnp.log(p_tok + eps) + g_tok
    s_msk = jnp.log(p_msk + eps) + g_msk

    z = jnp.where(is_mask_tok, _MASK, jnp.where(s_tok >= s_msk, ids, _MASK))
    z_ref[...] = z.astype(jnp.int32)


@functools.partial(jax.jit, static_argnames=("interpret",))
def kernel(input_ids, t, *, interpret=False):
    b, s = input_ids.shape
    n = b * s
    ids2 = input_ids.astype(jnp.int32).reshape(2, n // 2)
    t2 = jnp.broadcast_to(t.astype(jnp.float32)[:, None], (b, s)).reshape(2, n // 2)
    z = pl.pallas_call(
        _sample_body,
        out_shape=jax.ShapeDtypeStruct((2, n // 2), jnp.int32),
        interpret=interpret,
    )(ids2, t2)
    return z.reshape(b, s)


# speedup vs baseline: 7.8487x; 1.6784x over previous
import functools
import math

import jax
import jax.numpy as jnp
import numpy as np
from jax.experimental import pallas as pl

_VOCAB = 100000
_MASK = 99999
_GAMMA = 1.0
_P_UNIFORM = max(math.exp(-20.0), 0.0)
_LOG_B = max(
    _GAMMA * math.log(2.0) + math.log(_P_UNIFORM) - math.log(1.0 - _P_UNIFORM),
    -20.0,
)
_B_CONST = math.exp(_LOG_B)

_KEY_HI = np.uint32(0)
_KEY_LO = np.uint32(42)
_KS2 = np.uint32(int(_KEY_HI) ^ int(_KEY_LO) ^ 0x1BD11BDA)
_ROT_A = (13, 15, 26, 6)
_ROT_B = (17, 29, 16, 24)
_TINY = np.float32(np.finfo(np.float32).tiny)
_SPAN = np.float32(np.float32(1.0) - _TINY)


def _rotl(x, r):
    r = np.uint32(r)
    return (x << r) | (x >> np.uint32(32 - r))


def _threefry2x32(ctr_lo):
    ks = (_KEY_HI, _KEY_LO, _KS2)
    x0 = jnp.full_like(ctr_lo, ks[0])
    x1 = ctr_lo + ks[1]

    def rounds(x0, x1, rots):
        for r in rots:
            x0 = x0 + x1
            x1 = _rotl(x1, r)
            x1 = x0 ^ x1
        return x0, x1

    x0, x1 = rounds(x0, x1, _ROT_A)
    x0 = x0 + ks[1]
    x1 = x1 + ks[2] + np.uint32(1)
    x0, x1 = rounds(x0, x1, _ROT_B)
    x0 = x0 + ks[2]
    x1 = x1 + ks[0] + np.uint32(2)
    x0, x1 = rounds(x0, x1, _ROT_A)
    x0 = x0 + ks[0]
    x1 = x1 + ks[1] + np.uint32(3)
    x0, x1 = rounds(x0, x1, _ROT_B)
    x0 = x0 + ks[1]
    x1 = x1 + ks[2] + np.uint32(4)
    x0, x1 = rounds(x0, x1, _ROT_A)
    x0 = x0 + ks[2]
    x1 = x1 + ks[0] + np.uint32(5)
    return x0 ^ x1


def _gumbel_from_bits(bits):
    float_bits = (bits >> np.uint32(9)) | np.uint32(0x3F800000)
    floats = jax.lax.bitcast_convert_type(float_bits, jnp.float32) - jnp.float32(1.0)
    u = jnp.maximum(_TINY, floats * _SPAN + _TINY)
    return -jnp.log(-jnp.log(u))


def _sample_body(ids_ref, t_ref, z_ref):
    ids = ids_ref[...]
    t = jnp.broadcast_to(t_ref[...], ids.shape)

    row = (
        jax.lax.broadcasted_iota(jnp.int32, ids.shape, 0) * ids.shape[1]
        + jax.lax.broadcasted_iota(jnp.int32, ids.shape, 1)
    )
    base = row * _VOCAB
    bits_tok = _threefry2x32((base + ids).astype(jnp.uint32))
    bits_msk = _threefry2x32((base + _MASK).astype(jnp.uint32))
    g_tok = _gumbel_from_bits(bits_tok)
    g_msk = _gumbel_from_bits(bits_msk)

    one = jnp.float32(1.0)
    zero = jnp.float32(0.0)
    c_t = jnp.power(t, jnp.float32(0.5)) * jnp.power(one - t, jnp.float32(0.5)) * jnp.float32(_B_CONST)
    big_c = jnp.maximum(one + c_t, jnp.float32(1e-4))
    alpha = (one - t) / big_c
    unif = jnp.float32(np.float32(1.0) / np.float32(_VOCAB - 1))
    beta_tok = (t * zero + c_t * unif) / big_c
    beta_msk = (t * one + c_t * zero) / big_c

    is_mask_tok = ids == _MASK
    p_tok = jnp.where(is_mask_tok, one * alpha + beta_msk, one * alpha + beta_tok)
    p_msk = jnp.where(is_mask_tok, p_tok, zero * alpha + beta_msk)
    eps = jnp.float32(1e-12)
    s_tok = jnp.log(p_tok + eps) + g_tok
    s_msk = jnp.log(p_msk + eps) + g_msk

    z = jnp.where(is_mask_tok, _MASK, jnp.where(s_tok >= s_msk, ids, _MASK))
    z_ref[...] = z.astype(jnp.int32)


@functools.partial(jax.jit, static_argnames=("interpret",))
def kernel(input_ids, t, *, interpret=False):
    b, s = input_ids.shape
    return pl.pallas_call(
        _sample_body,
        out_shape=jax.ShapeDtypeStruct((b, s), jnp.int32),
        interpret=interpret,
    )(input_ids.astype(jnp.int32), t.astype(jnp.float32)[:, None])
